# Initial kernel scaffold; baseline (speedup 1.0000x reference)
import jax
import jax.numpy as jnp
from jax.experimental import pallas as pl


def kernel(x, edge_index, W1, b1, W2, b2):
    raise NotImplementedError("write your pallas kernel here")


# speedup vs baseline: 7.9982x; 7.9982x over previous
import functools

import jax
import jax.numpy as jnp
from jax import lax
from jax.experimental import pallas as pl
from jax.experimental.pallas import tpu as pltpu
from jax.experimental.pallas import tpu_sc as plsc

N_NODES = 10000
N_PAD = 10240
N_EDGES = 160000
N_TILES = 16
E_TILE = N_EDGES // N_TILES
CHUNK = 128
N_CHUNKS = (E_TILE + CHUNK - 1) // CHUNK
E_TILE_PAD = N_CHUNKS * CHUNK
E_TILE2 = N_EDGES // 32
N_CHUNKS2 = (E_TILE2 + CHUNK - 1) // CHUNK
E_TILE2_PAD = N_CHUNKS2 * CHUNK
ROWS_TILE = N_PAD // N_TILES
ROW_BLK = 512

_f32 = jnp.float32


def _mesh():
    return plsc.VectorSubcoreMesh(core_axis_name="c", subcore_axis_name="s")


@functools.partial(
    pl.kernel,
    out_type=jax.ShapeDtypeStruct((N_PAD, CHUNK), _f32),
    mesh=_mesh(),
    scratch_types=[
        pltpu.VMEM((N_CHUNKS, CHUNK), jnp.int32),
        pltpu.VMEM((CHUNK, CHUNK), _f32),
        pltpu.VMEM_SHARED((N_PAD, CHUNK), _f32),
        pltpu.SemaphoreType.DMA,
    ],
)
def _deg_kernel(dst_hbm, ones_hbm, zeros_hbm, deg_hbm, dst_v, ones_v, acc, sem):
    c = lax.axis_index("c")
    s = lax.axis_index("s")
    r0 = s * ROWS_TILE

    @pl.when(c == 0)
    def _():
        pltpu.sync_copy(dst_hbm.at[s], dst_v)
        pltpu.sync_copy(ones_hbm, ones_v)
        for k in range(ROWS_TILE // CHUNK):
            pltpu.sync_copy(zeros_hbm, acc.at[pl.ds(r0 + k * CHUNK, CHUNK)])
        plsc.subcore_barrier()

        def body(j, carry):
            pltpu.async_copy(ones_v, acc.at[dst_v.at[j]], sem, add=True).wait()
            return carry

        lax.fori_loop(0, N_CHUNKS, body, 0)
        plsc.subcore_barrier()
        pltpu.sync_copy(acc.at[pl.ds(r0, ROWS_TILE)], deg_hbm.at[pl.ds(r0, ROWS_TILE)])


@functools.partial(
    pl.kernel,
    out_type=jax.ShapeDtypeStruct((2 * N_PAD, 128), _f32),
    mesh=_mesh(),
    scratch_types=[
        pltpu.VMEM((N_CHUNKS, CHUNK), jnp.int32),
        pltpu.VMEM((N_CHUNKS, CHUNK), jnp.int32),
        pltpu.VMEM((CHUNK, 128), _f32),
        pltpu.VMEM_SHARED((N_PAD, 128), _f32),
        pltpu.SemaphoreType.DMA,
        pltpu.SemaphoreType.DMA,
    ],
)
def _agg_l1(src_hbm, dst_hbm, hs_hbm, out_hbm, src_v, dst_v, rows_v, acc, sem_g, sem_s):
    c = lax.axis_index("c")
    s = lax.axis_index("s")
    r0 = s * ROWS_TILE
    h0 = c * N_PAD

    pltpu.sync_copy(src_hbm.at[c * N_TILES + s], src_v)
    pltpu.sync_copy(dst_hbm.at[s], dst_v)
    pltpu.sync_copy(hs_hbm.at[pl.ds(h0 + r0, ROWS_TILE)], acc.at[pl.ds(r0, ROWS_TILE)])
    plsc.subcore_barrier()

    def body(j, carry):
        pltpu.async_copy(hs_hbm.at[src_v.at[j]], rows_v, sem_g).wait()
        pltpu.async_copy(rows_v, acc.at[dst_v.at[j]], sem_s, add=True).wait()
        return carry

    lax.fori_loop(0, N_CHUNKS, body, 0)
    plsc.subcore_barrier()

    pltpu.sync_copy(acc.at[pl.ds(r0, ROWS_TILE)], out_hbm.at[pl.ds(h0 + r0, ROWS_TILE)])


@functools.partial(
    pl.kernel,
    out_type=jax.ShapeDtypeStruct((2 * N_PAD, 128), _f32),
    mesh=_mesh(),
    scratch_types=[
        pltpu.VMEM((N_CHUNKS2, CHUNK), jnp.int32),
        pltpu.VMEM((N_CHUNKS2, CHUNK), jnp.int32),
        pltpu.VMEM((CHUNK, 128), _f32),
        pltpu.VMEM_SHARED((N_PAD, 128), _f32),
        pltpu.SemaphoreType.DMA,
        pltpu.SemaphoreType.DMA,
    ],
)
def _agg_l2(src_hbm, dst_hbm, hs_hbm, zeros_hbm, out_hbm,
            src_v, dst_v, rows_v, acc, sem_g, sem_s):
    c = lax.axis_index("c")
    s = lax.axis_index("s")
    t = c * N_TILES + s
    r0 = s * ROWS_TILE
    h0 = c * N_PAD

    pltpu.sync_copy(src_hbm.at[t], src_v)
    pltpu.sync_copy(dst_hbm.at[t], dst_v)
    for k in range(ROWS_TILE // CHUNK):
        pltpu.sync_copy(zeros_hbm, acc.at[pl.ds(r0 + k * CHUNK, CHUNK)])
    plsc.subcore_barrier()

    def body(j, carry):
        pltpu.async_copy(hs_hbm.at[src_v.at[j]], rows_v, sem_g).wait()
        pltpu.async_copy(rows_v, acc.at[dst_v.at[j]], sem_s, add=True).wait()
        return carry

    lax.fori_loop(0, N_CHUNKS2, body, 0)
    plsc.subcore_barrier()

    pltpu.sync_copy(acc.at[pl.ds(r0, ROWS_TILE)], out_hbm.at[pl.ds(h0 + r0, ROWS_TILE)])


def _tc_a_body(x_ref, w_ref, deg_ref, o_ref):
    dinv = lax.rsqrt(deg_ref[...] + 1.0)
    h = jnp.dot(x_ref[...], w_ref[...], preferred_element_type=_f32)
    o_ref[...] = (h * dinv)[None]


def _tc_b_body(a_ref, b_ref, deg_ref, w_ref, b1_ref, o_ref):
    dinv = lax.rsqrt(deg_ref[...] + 1.0)
    agg = jnp.concatenate([a_ref[...], b_ref[...]], axis=1)
    z = jnp.maximum(agg * dinv + b1_ref[...], 0.0)
    o_ref[...] = jnp.dot(z, w_ref[...], preferred_element_type=_f32) * dinv


def _tc_c_body(p0_ref, p1_ref, hs2_ref, deg_ref, b2_ref, o_ref):
    dinv = lax.rsqrt(deg_ref[...] + 1.0)
    agg = p0_ref[...] + p1_ref[...] + hs2_ref[...]
    o_ref[...] = agg * dinv + b2_ref[...]


def _row_spec(cols):
    return pl.BlockSpec((ROW_BLK, cols), lambda i: (i, 0))


def _full_spec(rows, cols):
    return pl.BlockSpec((rows, cols), lambda i: (0, 0))


_GRID = (N_PAD // ROW_BLK,)

_tc_a = pl.pallas_call(
    _tc_a_body,
    grid=(N_PAD // ROW_BLK, 2),
    in_specs=[pl.BlockSpec((ROW_BLK, 256), lambda i, j: (i, 0)),
              pl.BlockSpec((256, 128), lambda i, j: (0, j)),
              pl.BlockSpec((ROW_BLK, 1), lambda i, j: (i, 0))],
    out_specs=pl.BlockSpec((1, ROW_BLK, 128), lambda i, j: (j, i, 0)),
    out_shape=jax.ShapeDtypeStruct((2, N_PAD, 128), _f32),
)

_tc_b = pl.pallas_call(
    _tc_b_body,
    grid=_GRID,
    in_specs=[_row_spec(128), _row_spec(128), _row_spec(1),
              _full_spec(256, 128), _full_spec(1, 256)],
    out_specs=_row_spec(128),
    out_shape=jax.ShapeDtypeStruct((N_PAD, 128), _f32),
)

_tc_c = pl.pallas_call(
    _tc_c_body,
    grid=_GRID,
    in_specs=[_row_spec(128), _row_spec(128), _row_spec(128), _row_spec(1),
              _full_spec(1, 128)],
    out_specs=_row_spec(128),
    out_shape=jax.ShapeDtypeStruct((N_PAD, 128), _f32),
)


def kernel(x, edge_index, W1, b1, W2, b2):
    src = edge_index[0].astype(jnp.int32)
    dst = edge_index[1].astype(jnp.int32)
    src3 = jnp.pad(src.reshape(N_TILES, E_TILE), ((0, 0), (0, E_TILE_PAD - E_TILE)),
                   constant_values=N_NODES).reshape(N_TILES, N_CHUNKS, CHUNK)
    dst3 = jnp.pad(dst.reshape(N_TILES, E_TILE), ((0, 0), (0, E_TILE_PAD - E_TILE)),
                   constant_values=N_NODES).reshape(N_TILES, N_CHUNKS, CHUNK)

    src32 = jnp.pad(src.reshape(32, E_TILE2), ((0, 0), (0, E_TILE2_PAD - E_TILE2)),
                    constant_values=N_NODES).reshape(32, N_CHUNKS2, CHUNK)
    dst32 = jnp.pad(dst.reshape(32, E_TILE2), ((0, 0), (0, E_TILE2_PAD - E_TILE2)),
                    constant_values=N_NODES).reshape(32, N_CHUNKS2, CHUNK)

    ones128 = jnp.ones((CHUNK, CHUNK), _f32)
    zeros128 = jnp.zeros((CHUNK, CHUNK), _f32)
    degc = _deg_kernel(dst3, ones128, zeros128)[:, :1]

    srcl1 = jnp.concatenate([src3, src3 + N_PAD], axis=0)

    x_pad = jnp.pad(x, ((0, N_PAD - N_NODES), (0, 0)))
    hs1 = _tc_a(x_pad, W1, degc).reshape(2 * N_PAD, 128)
    agg1 = _agg_l1(srcl1, dst3, hs1).reshape(2, N_PAD, 128)
    hs2 = _tc_b(agg1[0], agg1[1], degc, W2, b1.reshape(1, 256))
    p = _agg_l2(src32, dst32, hs2, zeros128).reshape(2, N_PAD, 128)
    out = _tc_c(p[0], p[1], hs2, degc, b2.reshape(1, 128))
    return out[:N_NODES]


# speedup vs baseline: 8.2367x; 1.0298x over previous
import functools

import jax
import jax.numpy as jnp
from jax import lax
from jax.experimental import pallas as pl
from jax.experimental.pallas import tpu as pltpu
from jax.experimental.pallas import tpu_sc as plsc

N_NODES = 10000
N_PAD = 10240
N_EDGES = 160000
N_TILES = 16
CHUNK = 128
NBUF = 4
N_CHUNKS = 80
E_TILE = N_EDGES // N_TILES
E_TILE_PAD = N_CHUNKS * CHUNK
N_CHUNKS2 = 40
E_TILE2 = N_EDGES // 32
E_TILE2_PAD = N_CHUNKS2 * CHUNK
ROWS_TILE = N_PAD // N_TILES
ROW_BLK = 512

_f32 = jnp.float32


def _mesh():
    return plsc.VectorSubcoreMesh(core_axis_name="c", subcore_axis_name="s")


NIB1 = 16
NIB2 = 8


@functools.partial(
    pl.kernel,
    out_type=jax.ShapeDtypeStruct((2 * N_PAD, CHUNK), _f32),
    mesh=_mesh(),
    scratch_types=[
        pltpu.VMEM((N_CHUNKS2, CHUNK), jnp.int32),
        pltpu.VMEM((CHUNK, CHUNK), _f32),
        pltpu.VMEM_SHARED((N_PAD, CHUNK), _f32),
        pltpu.SemaphoreType.DMA((NBUF,)),
    ],
)
def _deg_kernel(dst_hbm, ones_hbm, zeros_hbm, deg_hbm, dst_v, ones_v, acc, sem):
    c = lax.axis_index("c")
    s = lax.axis_index("s")
    t = c * N_TILES + s
    r0 = s * ROWS_TILE
    h0 = c * N_PAD

    pltpu.sync_copy(dst_hbm.at[t], dst_v)
    pltpu.sync_copy(ones_hbm, ones_v)
    for k in range(ROWS_TILE // CHUNK):
        pltpu.sync_copy(zeros_hbm, acc.at[pl.ds(r0 + k * CHUNK, CHUNK)])
    plsc.subcore_barrier()

    def body(i, carry):
        for b in range(NBUF):
            pltpu.async_copy(ones_v, acc.at[dst_v.at[i * NBUF + b]], sem.at[b], add=True)
        for b in range(NBUF):
            pltpu.make_async_copy(ones_v, acc.at[dst_v.at[i * NBUF + b]], sem.at[b]).wait()
        return carry

    lax.fori_loop(0, N_CHUNKS2 // NBUF, body, 0)
    plsc.subcore_barrier()
    pltpu.sync_copy(acc.at[pl.ds(r0, ROWS_TILE)], deg_hbm.at[pl.ds(h0 + r0, ROWS_TILE)])


def _agg_body(src_hbm, dst_hbm, hs_hbm, out_hbm, sblk, dblk, rows, acc,
              sem_i, sem_g, sem_s, n_chunks, nib, init):
    c = lax.axis_index("c")
    s = lax.axis_index("s")
    t = c * N_TILES + s
    r0 = s * ROWS_TILE
    h0 = c * N_PAD
    n_blocks = n_chunks // nib

    def i_start(k, kb):
        pltpu.async_copy(src_hbm.at[t, pl.ds(k * nib, nib)], sblk.at[kb], sem_i.at[kb])
        pltpu.async_copy(dst_hbm.at[t, pl.ds(k * nib, nib)], dblk.at[kb], sem_i.at[kb])

    def i_wait(k, kb):
        pltpu.make_async_copy(src_hbm.at[t, pl.ds(k * nib, nib)], sblk.at[kb], sem_i.at[kb]).wait()
        pltpu.make_async_copy(dst_hbm.at[t, pl.ds(k * nib, nib)], dblk.at[kb], sem_i.at[kb]).wait()

    i_start(0, 0)
    init(c, r0, h0)
    plsc.subcore_barrier()

    def outer(k, carry):
        kb = lax.rem(k, 2)
        kn = lax.rem(k + 1, 2)
        i_wait(k, kb)

        @pl.when(k + 1 < n_blocks)
        def _():
            i_start(k + 1, kn)

        def g_start(q, b):
            pltpu.async_copy(hs_hbm.at[sblk.at[kb, q]], rows.at[b], sem_g.at[b])

        def g_wait(q, b):
            pltpu.make_async_copy(hs_hbm.at[sblk.at[kb, q]], rows.at[b], sem_g.at[b]).wait()

        def s_start(q, b):
            pltpu.async_copy(rows.at[b], acc.at[dblk.at[kb, q]], sem_s.at[b], add=True)

        def s_wait(q, b):
            pltpu.make_async_copy(rows.at[b], acc.at[dblk.at[kb, q]], sem_s.at[b]).wait()

        g_start(0, 0)

        def inner(q, carry2):
            b = lax.rem(q, 2)
            bn = lax.rem(q + 1, 2)

            @pl.when(q >= 1)
            def _():
                s_wait(q - 1, bn)

            @pl.when(q + 1 < nib)
            def _():
                g_start(q + 1, bn)

            g_wait(q, b)
            s_start(q, b)
            return carry2

        lax.fori_loop(0, nib, inner, 0)
        s_wait(nib - 1, (nib - 1) % 2)
        return carry

    lax.fori_loop(0, n_blocks, outer, 0)
    plsc.subcore_barrier()
    pltpu.sync_copy(acc.at[pl.ds(r0, ROWS_TILE)], out_hbm.at[pl.ds(h0 + r0, ROWS_TILE)])


def _agg_scratch(nib):
    return [
        pltpu.VMEM((2, nib, CHUNK), jnp.int32),
        pltpu.VMEM((2, nib, CHUNK), jnp.int32),
        pltpu.VMEM((2, CHUNK, CHUNK), _f32),
        pltpu.VMEM_SHARED((N_PAD, CHUNK), _f32),
        pltpu.SemaphoreType.DMA((2,)),
        pltpu.SemaphoreType.DMA((2,)),
        pltpu.SemaphoreType.DMA((2,)),
    ]


@functools.partial(
    pl.kernel,
    out_type=jax.ShapeDtypeStruct((2 * N_PAD, 128), _f32),
    mesh=_mesh(),
    scratch_types=_agg_scratch(NIB1),
)
def _agg_l1(src_hbm, dst_hbm, hs_hbm, out_hbm,
            sblk, dblk, rows, acc, sem_i, sem_g, sem_s):
    def init(c, r0, h0):
        pltpu.sync_copy(hs_hbm.at[pl.ds(h0 + r0, ROWS_TILE)], acc.at[pl.ds(r0, ROWS_TILE)])

    _agg_body(src_hbm, dst_hbm, hs_hbm, out_hbm, sblk, dblk, rows, acc,
              sem_i, sem_g, sem_s, N_CHUNKS, NIB1, init)


@functools.partial(
    pl.kernel,
    out_type=jax.ShapeDtypeStruct((2 * N_PAD, 128), _f32),
    mesh=_mesh(),
    scratch_types=_agg_scratch(NIB2),
)
def _agg_l2(src_hbm, dst_hbm, hs_hbm, zeros_hbm, out_hbm,
            sblk, dblk, rows, acc, sem_i, sem_g, sem_s):
    def init(c, r0, h0):
        for k in range(ROWS_TILE // CHUNK):
            pltpu.sync_copy(zeros_hbm, acc.at[pl.ds(r0 + k * CHUNK, CHUNK)])

    _agg_body(src_hbm, dst_hbm, hs_hbm, out_hbm, sblk, dblk, rows, acc,
              sem_i, sem_g, sem_s, N_CHUNKS2, NIB2, init)


def _tc_a_body(x_ref, w_ref, d0_ref, d1_ref, o_ref):
    dinv = lax.rsqrt(d0_ref[...] + d1_ref[...] + 1.0)
    h = jnp.dot(x_ref[...], w_ref[...], preferred_element_type=_f32)
    o_ref[...] = (h * dinv)[None]


def _tc_b_body(a_ref, b_ref, d0_ref, d1_ref, w_ref, b1_ref, o_ref):
    dinv = lax.rsqrt(d0_ref[...] + d1_ref[...] + 1.0)
    agg = jnp.concatenate([a_ref[...], b_ref[...]], axis=1)
    z = jnp.maximum(agg * dinv + b1_ref[...], 0.0)
    o_ref[...] = jnp.dot(z, w_ref[...], preferred_element_type=_f32) * dinv


def _tc_c_body(p0_ref, p1_ref, hs2_ref, d0_ref, d1_ref, b2_ref, o_ref):
    dinv = lax.rsqrt(d0_ref[...] + d1_ref[...] + 1.0)
    agg = p0_ref[...] + p1_ref[...] + hs2_ref[...]
    o_ref[...] = agg * dinv + b2_ref[...]


def _row_spec(cols):
    return pl.BlockSpec((ROW_BLK, cols), lambda i: (i, 0))


def _full_spec(rows, cols):
    return pl.BlockSpec((rows, cols), lambda i: (0, 0))


_GRID = (N_PAD // ROW_BLK,)

_tc_a = pl.pallas_call(
    _tc_a_body,
    grid=(N_PAD // ROW_BLK, 2),
    in_specs=[pl.BlockSpec((ROW_BLK, 256), lambda i, j: (i, 0)),
              pl.BlockSpec((256, 128), lambda i, j: (0, j)),
              pl.BlockSpec((ROW_BLK, 1), lambda i, j: (i, 0)),
              pl.BlockSpec((ROW_BLK, 1), lambda i, j: (i, 0))],
    out_specs=pl.BlockSpec((1, ROW_BLK, 128), lambda i, j: (j, i, 0)),
    out_shape=jax.ShapeDtypeStruct((2, N_PAD, 128), _f32),
)

_tc_b = pl.pallas_call(
    _tc_b_body,
    grid=_GRID,
    in_specs=[_row_spec(128), _row_spec(128), _row_spec(1), _row_spec(1),
              _full_spec(256, 128), _full_spec(1, 256)],
    out_specs=_row_spec(128),
    out_shape=jax.ShapeDtypeStruct((N_PAD, 128), _f32),
)

_tc_c = pl.pallas_call(
    _tc_c_body,
    grid=_GRID,
    in_specs=[_row_spec(128), _row_spec(128), _row_spec(128), _row_spec(1),
              _row_spec(1), _full_spec(1, 128)],
    out_specs=_row_spec(128),
    out_shape=jax.ShapeDtypeStruct((N_PAD, 128), _f32),
)


def _pad_edges(v, n_split, e_tile, e_pad, n_chunks):
    return jnp.pad(v.reshape(n_split, e_tile), ((0, 0), (0, e_pad - e_tile)),
                   constant_values=N_NODES).reshape(n_split, n_chunks, CHUNK)


def kernel(x, edge_index, W1, b1, W2, b2):
    src = edge_index[0].astype(jnp.int32)
    dst = edge_index[1].astype(jnp.int32)
    src3 = _pad_edges(src, N_TILES, E_TILE, E_TILE_PAD, N_CHUNKS)
    dst3 = _pad_edges(dst, N_TILES, E_TILE, E_TILE_PAD, N_CHUNKS)
    srcl1 = jnp.concatenate([src3, src3 + N_PAD], axis=0)
    dstl1 = jnp.concatenate([dst3, dst3], axis=0)
    src32 = _pad_edges(src, 32, E_TILE2, E_TILE2_PAD, N_CHUNKS2)
    dst32 = _pad_edges(dst, 32, E_TILE2, E_TILE2_PAD, N_CHUNKS2)

    ones128 = jnp.ones((CHUNK, CHUNK), _f32)
    zeros128 = jnp.zeros((CHUNK, CHUNK), _f32)
    degp = _deg_kernel(dst32, ones128, zeros128)
    d0 = degp[:N_PAD, :1]
    d1 = degp[N_PAD:, :1]

    x_pad = jnp.pad(x, ((0, N_PAD - N_NODES), (0, 0)))
    hs1 = _tc_a(x_pad, W1, d0, d1).reshape(2 * N_PAD, 128)
    agg1 = _agg_l1(srcl1, dstl1, hs1).reshape(2, N_PAD, 128)
    hs2 = _tc_b(agg1[0], agg1[1], d0, d1, W2, b1.reshape(1, 256))
    p = _agg_l2(src32, dst32, hs2, zeros128).reshape(2, N_PAD, 128)
    out = _tc_c(p[0], p[1], hs2, d0, d1, b2.reshape(1, 128))
    return out[:N_NODES]


# speedup vs baseline: 9.1857x; 1.1152x over previous
import functools

import jax
import jax.numpy as jnp
from jax import lax
from jax.experimental import pallas as pl
from jax.experimental.pallas import tpu as pltpu
from jax.experimental.pallas import tpu_sc as plsc

N_NODES = 10000
N_PAD = 10240
N_EDGES = 160000
N_TILES = 16
CHUNK = 128
NBUF = 4
N_CHUNKS = 80
E_TILE = N_EDGES // N_TILES
E_TILE_PAD = N_CHUNKS * CHUNK
N_CHUNKS2 = 40
E_TILE2 = N_EDGES // 32
E_TILE2_PAD = N_CHUNKS2 * CHUNK
ROWS_TILE = N_PAD // N_TILES
ROW_BLK = 512

_f32 = jnp.float32


def _mesh():
    return plsc.VectorSubcoreMesh(core_axis_name="c", subcore_axis_name="s")


NIB1 = 16
NIB2 = 8


@functools.partial(
    pl.kernel,
    out_type=jax.ShapeDtypeStruct((2 * N_PAD, CHUNK), _f32),
    mesh=_mesh(),
    scratch_types=[
        pltpu.VMEM((N_CHUNKS2, CHUNK), jnp.int32),
        pltpu.VMEM((CHUNK, CHUNK), _f32),
        pltpu.VMEM_SHARED((N_PAD, CHUNK), _f32),
        pltpu.SemaphoreType.DMA((NBUF,)),
    ],
)
def _deg_kernel(dst_hbm, ones_hbm, zeros_hbm, deg_hbm, dst_v, ones_v, acc, sem):
    c = lax.axis_index("c")
    s = lax.axis_index("s")
    t = c * N_TILES + s
    r0 = s * ROWS_TILE
    h0 = c * N_PAD

    pltpu.sync_copy(dst_hbm.at[t], dst_v)
    pltpu.sync_copy(ones_hbm, ones_v)
    for k in range(ROWS_TILE // CHUNK):
        pltpu.sync_copy(zeros_hbm, acc.at[pl.ds(r0 + k * CHUNK, CHUNK)])
    plsc.subcore_barrier()

    def body(i, carry):
        for b in range(NBUF):
            pltpu.async_copy(ones_v, acc.at[dst_v.at[i * NBUF + b]], sem.at[b], add=True)
        for b in range(NBUF):
            pltpu.make_async_copy(ones_v, acc.at[dst_v.at[i * NBUF + b]], sem.at[b]).wait()
        return carry

    lax.fori_loop(0, N_CHUNKS2 // NBUF, body, 0)
    plsc.subcore_barrier()
    pltpu.sync_copy(acc.at[pl.ds(r0, ROWS_TILE)], deg_hbm.at[pl.ds(h0 + r0, ROWS_TILE)])


def _agg_body(src_hbm, dst_hbm, hs_hbm, out_hbm, sblk, dblk, rows, acc,
              sem_i, sem_g, sem_s, n_chunks, nib, init):
    c = lax.axis_index("c")
    s = lax.axis_index("s")
    t = c * N_TILES + s
    r0 = s * ROWS_TILE
    h0 = c * N_PAD
    n_blocks = n_chunks // nib

    def i_start(k, kb):
        pltpu.async_copy(src_hbm.at[t, pl.ds(k * nib, nib)], sblk.at[kb], sem_i.at[kb])
        pltpu.async_copy(dst_hbm.at[t, pl.ds(k * nib, nib)], dblk.at[kb], sem_i.at[kb])

    def i_wait(k, kb):
        pltpu.make_async_copy(src_hbm.at[t, pl.ds(k * nib, nib)], sblk.at[kb], sem_i.at[kb]).wait()
        pltpu.make_async_copy(dst_hbm.at[t, pl.ds(k * nib, nib)], dblk.at[kb], sem_i.at[kb]).wait()

    i_start(0, 0)
    init(c, r0, h0)
    plsc.subcore_barrier()

    def outer(k, carry):
        kb = lax.rem(k, 2)
        kn = lax.rem(k + 1, 2)
        i_wait(k, kb)

        @pl.when(k + 1 < n_blocks)
        def _():
            i_start(k + 1, kn)

        def g_start(q, b):
            pltpu.async_copy(hs_hbm.at[sblk.at[kb, q]], rows.at[b], sem_g.at[b])

        def g_wait(q, b):
            pltpu.make_async_copy(hs_hbm.at[sblk.at[kb, q]], rows.at[b], sem_g.at[b]).wait()

        def s_start(q, b):
            pltpu.async_copy(rows.at[b], acc.at[dblk.at[kb, q]], sem_s.at[b], add=True)

        def s_wait(q, b):
            pltpu.make_async_copy(rows.at[b], acc.at[dblk.at[kb, q]], sem_s.at[b]).wait()

        g_start(0, 0)

        def inner(q, carry2):
            b = lax.rem(q, 2)
            bn = lax.rem(q + 1, 2)

            @pl.when(q >= 1)
            def _():
                s_wait(q - 1, bn)

            @pl.when(q + 1 < nib)
            def _():
                g_start(q + 1, bn)

            g_wait(q, b)
            s_start(q, b)
            return carry2

        lax.fori_loop(0, nib, inner, 0)
        s_wait(nib - 1, (nib - 1) % 2)
        return carry

    lax.fori_loop(0, n_blocks, outer, 0)
    plsc.subcore_barrier()
    pltpu.sync_copy(acc.at[pl.ds(r0, ROWS_TILE)], out_hbm.at[pl.ds(h0 + r0, ROWS_TILE)])


def _agg_scratch(nib):
    return [
        pltpu.VMEM((2, nib, CHUNK), jnp.int32),
        pltpu.VMEM((2, nib, CHUNK), jnp.int32),
        pltpu.VMEM((2, CHUNK, CHUNK), _f32),
        pltpu.VMEM_SHARED((N_PAD, CHUNK), _f32),
        pltpu.SemaphoreType.DMA((2,)),
        pltpu.SemaphoreType.DMA((2,)),
        pltpu.SemaphoreType.DMA((2,)),
    ]


@functools.partial(
    pl.kernel,
    out_type=jax.ShapeDtypeStruct((2 * N_PAD, 128), _f32),
    mesh=_mesh(),
    scratch_types=_agg_scratch(NIB1),
)
def _agg_l1(src_hbm, dst_hbm, hs_hbm, out_hbm,
            sblk, dblk, rows, acc, sem_i, sem_g, sem_s):
    def init(c, r0, h0):
        pltpu.sync_copy(hs_hbm.at[pl.ds(h0 + r0, ROWS_TILE)], acc.at[pl.ds(r0, ROWS_TILE)])

    _agg_body(src_hbm, dst_hbm, hs_hbm, out_hbm, sblk, dblk, rows, acc,
              sem_i, sem_g, sem_s, N_CHUNKS, NIB1, init)


@functools.partial(
    pl.kernel,
    out_type=jax.ShapeDtypeStruct((2 * N_PAD, 128), _f32),
    mesh=_mesh(),
    scratch_types=_agg_scratch(NIB2),
)
def _agg_l2(src_hbm, dst_hbm, hs_hbm, zeros_hbm, out_hbm,
            sblk, dblk, rows, acc, sem_i, sem_g, sem_s):
    def init(c, r0, h0):
        for k in range(ROWS_TILE // CHUNK):
            pltpu.sync_copy(zeros_hbm, acc.at[pl.ds(r0 + k * CHUNK, CHUNK)])

    _agg_body(src_hbm, dst_hbm, hs_hbm, out_hbm, sblk, dblk, rows, acc,
              sem_i, sem_g, sem_s, N_CHUNKS2, NIB2, init)


def _tc_a_body(x_ref, w_ref, d0_ref, d1_ref, o_ref):
    dinv = lax.rsqrt(d0_ref[...] + d1_ref[...] + 1.0)
    h = jnp.dot(x_ref[...], w_ref[...], preferred_element_type=_f32)
    o_ref[...] = (h * dinv)[None]


def _tc_b_body(a_ref, b_ref, d0_ref, d1_ref, w_ref, b1_ref, o_ref):
    dinv = lax.rsqrt(d0_ref[...] + d1_ref[...] + 1.0)
    agg = jnp.concatenate([a_ref[...], b_ref[...]], axis=1)
    z = jnp.maximum(agg * dinv + b1_ref[...], 0.0)
    hs2 = jnp.dot(z, w_ref[...], preferred_element_type=_f32) * dinv
    o_ref[...] = jnp.broadcast_to(hs2[None], (2,) + hs2.shape)


def _tc_c_body(p0_ref, p1_ref, hs2_ref, d0_ref, d1_ref, b2_ref, o_ref):
    dinv = lax.rsqrt(d0_ref[...] + d1_ref[...] + 1.0)
    agg = p0_ref[...] + p1_ref[...] + hs2_ref[...]
    o_ref[...] = agg * dinv + b2_ref[...]


def _row_spec(cols):
    return pl.BlockSpec((ROW_BLK, cols), lambda i: (i, 0))


def _full_spec(rows, cols):
    return pl.BlockSpec((rows, cols), lambda i: (0, 0))


_GRID = (N_PAD // ROW_BLK,)

_tc_a = pl.pallas_call(
    _tc_a_body,
    grid=(N_PAD // ROW_BLK, 2),
    in_specs=[pl.BlockSpec((ROW_BLK, 256), lambda i, j: (i, 0)),
              pl.BlockSpec((256, 128), lambda i, j: (0, j)),
              pl.BlockSpec((ROW_BLK, 1), lambda i, j: (i, 0)),
              pl.BlockSpec((ROW_BLK, 1), lambda i, j: (i, 0))],
    out_specs=pl.BlockSpec((1, ROW_BLK, 128), lambda i, j: (j, i, 0)),
    out_shape=jax.ShapeDtypeStruct((2, N_PAD, 128), _f32),
)

_tc_b = pl.pallas_call(
    _tc_b_body,
    grid=_GRID,
    in_specs=[_row_spec(128), _row_spec(128), _row_spec(1), _row_spec(1),
              _full_spec(256, 128), _full_spec(1, 256)],
    out_specs=pl.BlockSpec((2, ROW_BLK, 128), lambda i: (0, i, 0)),
    out_shape=jax.ShapeDtypeStruct((2, N_PAD, 128), _f32),
)

_tc_c = pl.pallas_call(
    _tc_c_body,
    grid=_GRID,
    in_specs=[_row_spec(128), _row_spec(128), _row_spec(128), _row_spec(1),
              _row_spec(1), _full_spec(1, 128)],
    out_specs=_row_spec(128),
    out_shape=jax.ShapeDtypeStruct((N_PAD, 128), _f32),
)


def _pad_edges(v, n_split, e_tile, e_pad, n_chunks):
    return jnp.pad(v.reshape(n_split, e_tile), ((0, 0), (0, e_pad - e_tile)),
                   constant_values=N_NODES).reshape(n_split, n_chunks, CHUNK)


def kernel(x, edge_index, W1, b1, W2, b2):
    src = edge_index[0].astype(jnp.int32)
    dst = edge_index[1].astype(jnp.int32)
    src3 = _pad_edges(src, N_TILES, E_TILE, E_TILE_PAD, N_CHUNKS)
    dst3 = _pad_edges(dst, N_TILES, E_TILE, E_TILE_PAD, N_CHUNKS)
    srcl1 = jnp.concatenate([src3, src3 + N_PAD], axis=0)
    dstl1 = jnp.concatenate([dst3, dst3], axis=0)
    src32 = _pad_edges(src, 32, E_TILE2, E_TILE2_PAD, N_CHUNKS2)
    dst32 = _pad_edges(dst, 32, E_TILE2, E_TILE2_PAD, N_CHUNKS2)
    src32o = jnp.concatenate([src32[:N_TILES], src32[N_TILES:] + N_PAD], axis=0)

    ones128 = jnp.ones((CHUNK, CHUNK), _f32)
    zeros128 = jnp.zeros((CHUNK, CHUNK), _f32)
    degp = _deg_kernel(dst32, ones128, zeros128)
    d0 = degp[:N_PAD, :1]
    d1 = degp[N_PAD:, :1]

    x_pad = jnp.pad(x, ((0, N_PAD - N_NODES), (0, 0)))
    hs1 = _tc_a(x_pad, W1, d0, d1).reshape(2 * N_PAD, 128)
    agg1 = _agg_l1(srcl1, dstl1, hs1).reshape(2, N_PAD, 128)
    hs2 = _tc_b(agg1[0], agg1[1], d0, d1, W2, b1.reshape(1, 256))
    hs2cat = hs2.reshape(2 * N_PAD, 128)
    p = _agg_l2(src32o, dst32, hs2cat, zeros128).reshape(2, N_PAD, 128)
    out = _tc_c(p[0], p[1], hs2[0], d0, d1, b2.reshape(1, 128))
    return out[:N_NODES]


# speedup vs baseline: 9.7195x; 1.0581x over previous
import functools

import jax
import jax.numpy as jnp
from jax import lax
from jax.experimental import pallas as pl
from jax.experimental.pallas import tpu as pltpu
from jax.experimental.pallas import tpu_sc as plsc

N_NODES = 10000
N_PAD = 10240
N_EDGES = 160000
N_TILES = 16
CHUNK = 128
NBUF = 4
N_CHUNKS = 80
E_TILE = N_EDGES // N_TILES
E_TILE_PAD = N_CHUNKS * CHUNK
N_CHUNKS2 = 40
E_TILE2 = N_EDGES // 32
E_TILE2_PAD = N_CHUNKS2 * CHUNK
ROWS_TILE = N_PAD // N_TILES
ROW_BLK = 512

_f32 = jnp.float32


def _mesh():
    return plsc.VectorSubcoreMesh(core_axis_name="c", subcore_axis_name="s")


NIB1 = 16
NIB2 = 8


@functools.partial(
    pl.kernel,
    out_type=jax.ShapeDtypeStruct((2 * N_PAD, CHUNK), _f32),
    mesh=_mesh(),
    scratch_types=[
        pltpu.VMEM((N_CHUNKS2, CHUNK), jnp.int32),
        pltpu.VMEM((CHUNK, CHUNK), _f32),
        pltpu.VMEM_SHARED((N_PAD, CHUNK), _f32),
        pltpu.SemaphoreType.DMA((NBUF,)),
    ],
)
def _deg_kernel(dst_hbm, ones_hbm, zeros_hbm, deg_hbm, dst_v, ones_v, acc, sem):
    c = lax.axis_index("c")
    s = lax.axis_index("s")
    t = c * N_TILES + s
    r0 = s * ROWS_TILE
    h0 = c * N_PAD

    pltpu.sync_copy(dst_hbm.at[t], dst_v)
    pltpu.sync_copy(ones_hbm, ones_v)
    for k in range(ROWS_TILE // CHUNK):
        pltpu.sync_copy(zeros_hbm, acc.at[pl.ds(r0 + k * CHUNK, CHUNK)])
    plsc.subcore_barrier()

    def body(i, carry):
        for b in range(NBUF):
            pltpu.async_copy(ones_v, acc.at[dst_v.at[i * NBUF + b]], sem.at[b], add=True)
        for b in range(NBUF):
            pltpu.make_async_copy(ones_v, acc.at[dst_v.at[i * NBUF + b]], sem.at[b]).wait()
        return carry

    lax.fori_loop(0, N_CHUNKS2 // NBUF, body, 0)
    plsc.subcore_barrier()
    pltpu.sync_copy(acc.at[pl.ds(r0, ROWS_TILE)], deg_hbm.at[pl.ds(h0 + r0, ROWS_TILE)])


def _agg_body(src_hbm, dst_hbm, hs_hbm, out_hbm, sblk, dblk, rows, acc,
              sem_i, sem_g, sem_s, n_chunks, nib, init):
    c = lax.axis_index("c")
    s = lax.axis_index("s")
    t = c * N_TILES + s
    r0 = s * ROWS_TILE
    h0 = c * N_PAD
    n_blocks = n_chunks // nib

    def i_start(k, kb):
        pltpu.async_copy(src_hbm.at[t, pl.ds(k * nib, nib)], sblk.at[kb], sem_i.at[kb])
        pltpu.async_copy(dst_hbm.at[t, pl.ds(k * nib, nib)], dblk.at[kb], sem_i.at[kb])

    def i_wait(k, kb):
        pltpu.make_async_copy(src_hbm.at[t, pl.ds(k * nib, nib)], sblk.at[kb], sem_i.at[kb]).wait()
        pltpu.make_async_copy(dst_hbm.at[t, pl.ds(k * nib, nib)], dblk.at[kb], sem_i.at[kb]).wait()

    i_start(0, 0)
    init(c, r0, h0)
    plsc.subcore_barrier()

    def outer(k, carry):
        kb = lax.rem(k, 2)
        kn = lax.rem(k + 1, 2)
        i_wait(k, kb)

        @pl.when(k + 1 < n_blocks)
        def _():
            i_start(k + 1, kn)

        def g_start(q, b):
            pltpu.async_copy(hs_hbm.at[sblk.at[kb, q]], rows.at[b], sem_g.at[b])

        def g_wait(q, b):
            pltpu.make_async_copy(hs_hbm.at[sblk.at[kb, q]], rows.at[b], sem_g.at[b]).wait()

        def s_start(q, b):
            pltpu.async_copy(rows.at[b], acc.at[dblk.at[kb, q]], sem_s.at[b], add=True)

        def s_wait(q, b):
            pltpu.make_async_copy(rows.at[b], acc.at[dblk.at[kb, q]], sem_s.at[b]).wait()

        g_start(0, 0)

        def inner(q, carry2):
            b = lax.rem(q, 2)
            bn = lax.rem(q + 1, 2)

            @pl.when(q >= 1)
            def _():
                s_wait(q - 1, bn)

            @pl.when(q + 1 < nib)
            def _():
                g_start(q + 1, bn)

            g_wait(q, b)
            s_start(q, b)
            return carry2

        lax.fori_loop(0, nib, inner, 0)
        s_wait(nib - 1, (nib - 1) % 2)
        return carry

    lax.fori_loop(0, n_blocks, outer, 0)
    plsc.subcore_barrier()
    pltpu.sync_copy(acc.at[pl.ds(r0, ROWS_TILE)], out_hbm.at[pl.ds(h0 + r0, ROWS_TILE)])


def _agg_scratch(nib):
    return [
        pltpu.VMEM((2, nib, CHUNK), jnp.int32),
        pltpu.VMEM((2, nib, CHUNK), jnp.int32),
        pltpu.VMEM((2, CHUNK, CHUNK), _f32),
        pltpu.VMEM_SHARED((N_PAD, CHUNK), _f32),
        pltpu.SemaphoreType.DMA((2,)),
        pltpu.SemaphoreType.DMA((2,)),
        pltpu.SemaphoreType.DMA((2,)),
    ]


@functools.partial(
    pl.kernel,
    out_type=jax.ShapeDtypeStruct((2 * N_PAD, 128), _f32),
    mesh=_mesh(),
    scratch_types=_agg_scratch(NIB1),
)
def _agg_l1(src_hbm, dst_hbm, hs_hbm, out_hbm,
            sblk, dblk, rows, acc, sem_i, sem_g, sem_s):
    def init(c, r0, h0):
        pltpu.sync_copy(hs_hbm.at[pl.ds(h0 + r0, ROWS_TILE)], acc.at[pl.ds(r0, ROWS_TILE)])

    _agg_body(src_hbm, dst_hbm, hs_hbm, out_hbm, sblk, dblk, rows, acc,
              sem_i, sem_g, sem_s, N_CHUNKS, NIB1, init)


@functools.partial(
    pl.kernel,
    out_type=jax.ShapeDtypeStruct((2 * N_PAD, 128), _f32),
    mesh=_mesh(),
    scratch_types=_agg_scratch(NIB2),
)
def _agg_l2(src_hbm, dst_hbm, hs_hbm, zeros_hbm, out_hbm,
            sblk, dblk, rows, acc, sem_i, sem_g, sem_s):
    def init(c, r0, h0):
        for k in range(ROWS_TILE // CHUNK):
            pltpu.sync_copy(zeros_hbm, acc.at[pl.ds(r0 + k * CHUNK, CHUNK)])

    _agg_body(src_hbm, dst_hbm, hs_hbm, out_hbm, sblk, dblk, rows, acc,
              sem_i, sem_g, sem_s, N_CHUNKS2, NIB2, init)


def _tc_a_body(x_ref, w_ref, d0_ref, d1_ref, o_ref):
    dinv = lax.rsqrt(d0_ref[...] + d1_ref[...] + 1.0)
    h = jnp.dot(x_ref[...], w_ref[...], preferred_element_type=_f32)
    o_ref[...] = (h * dinv)[None]


def _tc_b_body(a_ref, b_ref, d0_ref, d1_ref, w_ref, b1_ref, o_ref):
    dinv = lax.rsqrt(d0_ref[...] + d1_ref[...] + 1.0)
    agg = jnp.concatenate([a_ref[...], b_ref[...]], axis=1)
    z = jnp.maximum(agg * dinv + b1_ref[...], 0.0)
    hs2 = jnp.dot(z, w_ref[...], preferred_element_type=_f32) * dinv
    o_ref[...] = jnp.broadcast_to(hs2[None], (2,) + hs2.shape)


def _tc_c_body(p0_ref, p1_ref, hs2_ref, d0_ref, d1_ref, b2_ref, o_ref):
    dinv = lax.rsqrt(d0_ref[...] + d1_ref[...] + 1.0)
    agg = p0_ref[...] + p1_ref[...] + hs2_ref[...]
    o_ref[...] = agg * dinv + b2_ref[...]


def _row_spec(cols):
    return pl.BlockSpec((ROW_BLK, cols), lambda i: (i, 0))


def _full_spec(rows, cols):
    return pl.BlockSpec((rows, cols), lambda i: (0, 0))


_GRID = (N_PAD // ROW_BLK,)

_tc_a = pl.pallas_call(
    _tc_a_body,
    grid=(N_PAD // ROW_BLK, 2),
    in_specs=[pl.BlockSpec((ROW_BLK, 256), lambda i, j: (i, 0)),
              pl.BlockSpec((256, 128), lambda i, j: (0, j)),
              pl.BlockSpec((ROW_BLK, 1), lambda i, j: (i, 0)),
              pl.BlockSpec((ROW_BLK, 1), lambda i, j: (i + N_PAD // ROW_BLK, 0))],
    out_specs=pl.BlockSpec((1, ROW_BLK, 128), lambda i, j: (j, i, 0)),
    out_shape=jax.ShapeDtypeStruct((2, N_PAD, 128), _f32),
)

_NB = N_PAD // ROW_BLK

_tc_b = pl.pallas_call(
    _tc_b_body,
    grid=_GRID,
    in_specs=[pl.BlockSpec((ROW_BLK, 128), lambda i: (i, 0)),
              pl.BlockSpec((ROW_BLK, 128), lambda i: (i + _NB, 0)),
              pl.BlockSpec((ROW_BLK, 1), lambda i: (i, 0)),
              pl.BlockSpec((ROW_BLK, 1), lambda i: (i + _NB, 0)),
              _full_spec(256, 128), _full_spec(1, 256)],
    out_specs=pl.BlockSpec((2, ROW_BLK, 128), lambda i: (0, i, 0)),
    out_shape=jax.ShapeDtypeStruct((2, N_PAD, 128), _f32),
)

_tc_c = pl.pallas_call(
    _tc_c_body,
    grid=_GRID,
    in_specs=[pl.BlockSpec((ROW_BLK, 128), lambda i: (i, 0)),
              pl.BlockSpec((ROW_BLK, 128), lambda i: (i + _NB, 0)),
              pl.BlockSpec((ROW_BLK, 128), lambda i: (i, 0)),
              pl.BlockSpec((ROW_BLK, 1), lambda i: (i, 0)),
              pl.BlockSpec((ROW_BLK, 1), lambda i: (i + _NB, 0)),
              _full_spec(1, 128)],
    out_specs=_row_spec(128),
    out_shape=jax.ShapeDtypeStruct((N_PAD, 128), _f32),
)


def _pad_edges(v, n_split, e_tile, e_pad, n_chunks):
    return jnp.pad(v.reshape(n_split, e_tile), ((0, 0), (0, e_pad - e_tile)),
                   constant_values=N_NODES).reshape(n_split, n_chunks, CHUNK)


def kernel(x, edge_index, W1, b1, W2, b2):
    src = edge_index[0].astype(jnp.int32)
    dst = edge_index[1].astype(jnp.int32)
    src3 = _pad_edges(src, N_TILES, E_TILE, E_TILE_PAD, N_CHUNKS)
    dst3 = _pad_edges(dst, N_TILES, E_TILE, E_TILE_PAD, N_CHUNKS)
    srcl1 = jnp.concatenate([src3, src3 + N_PAD], axis=0)
    dstl1 = jnp.concatenate([dst3, dst3], axis=0)
    src32 = _pad_edges(src, 32, E_TILE2, E_TILE2_PAD, N_CHUNKS2)
    dst32 = _pad_edges(dst, 32, E_TILE2, E_TILE2_PAD, N_CHUNKS2)
    src32o = jnp.concatenate([src32[:N_TILES], src32[N_TILES:] + N_PAD], axis=0)

    ones128 = jnp.ones((CHUNK, CHUNK), _f32)
    zeros128 = jnp.zeros((CHUNK, CHUNK), _f32)
    degc = _deg_kernel(dst32, ones128, zeros128)[:, :1]

    hs1 = _tc_a(x, W1, degc, degc).reshape(2 * N_PAD, 128)
    agg1 = _agg_l1(srcl1, dstl1, hs1)
    hs2cat = _tc_b(agg1, agg1, degc, degc, W2, b1.reshape(1, 256)).reshape(2 * N_PAD, 128)
    p = _agg_l2(src32o, dst32, hs2cat, zeros128)
    out = _tc_c(p, p, hs2cat, degc, degc, b2.reshape(1, 128))
    return out[:N_NODES]


# speedup vs baseline: 9.8054x; 1.0088x over previous
import functools

import jax
import jax.numpy as jnp
from jax import lax
from jax.experimental import pallas as pl
from jax.experimental.pallas import tpu as pltpu
from jax.experimental.pallas import tpu_sc as plsc

N_NODES = 10000
N_PAD = 10240
N_EDGES = 160000
N_TILES = 16
CHUNK = 128
NBUF = 4
N_CHUNKS = 80
E_TILE = N_EDGES // N_TILES
E_TILE_PAD = N_CHUNKS * CHUNK
N_CHUNKS2 = 40
E_TILE2 = N_EDGES // 32
E_TILE2_PAD = N_CHUNKS2 * CHUNK
ROWS_TILE = N_PAD // N_TILES
ROW_BLK = 512

_f32 = jnp.float32


def _mesh():
    return plsc.VectorSubcoreMesh(core_axis_name="c", subcore_axis_name="s")


NIB1 = 16
NIB2 = 8


@functools.partial(
    pl.kernel,
    out_type=jax.ShapeDtypeStruct((2 * N_PAD, CHUNK), _f32),
    mesh=_mesh(),
    scratch_types=[
        pltpu.VMEM((N_CHUNKS2, CHUNK), jnp.int32),
        pltpu.VMEM((CHUNK, CHUNK), _f32),
        pltpu.VMEM_SHARED((N_PAD, CHUNK), _f32),
        pltpu.SemaphoreType.DMA((NBUF,)),
    ],
)
def _deg_kernel(dst_hbm, ones_hbm, zeros_hbm, deg_hbm, dst_v, ones_v, acc, sem):
    c = lax.axis_index("c")
    s = lax.axis_index("s")
    t = c * N_TILES + s
    r0 = s * ROWS_TILE
    h0 = c * N_PAD

    pltpu.sync_copy(dst_hbm.at[t], dst_v)
    pltpu.sync_copy(ones_hbm, ones_v)
    for k in range(ROWS_TILE // CHUNK):
        pltpu.sync_copy(zeros_hbm, acc.at[pl.ds(r0 + k * CHUNK, CHUNK)])
    plsc.subcore_barrier()

    def body(i, carry):
        for b in range(NBUF):
            pltpu.async_copy(ones_v, acc.at[dst_v.at[i * NBUF + b]], sem.at[b], add=True)
        for b in range(NBUF):
            pltpu.make_async_copy(ones_v, acc.at[dst_v.at[i * NBUF + b]], sem.at[b]).wait()
        return carry

    lax.fori_loop(0, N_CHUNKS2 // NBUF, body, 0)
    plsc.subcore_barrier()
    pltpu.sync_copy(acc.at[pl.ds(r0, ROWS_TILE)], deg_hbm.at[pl.ds(h0 + r0, ROWS_TILE)])


def _agg_body(src_hbm, dst_hbm, hs_hbm, out_hbm, sblk, dblk, rows, acc,
              sem_i, sem_g, sem_s, n_chunks, nib, init):
    c = lax.axis_index("c")
    s = lax.axis_index("s")
    t = c * N_TILES + s
    r0 = s * ROWS_TILE
    h0 = c * N_PAD
    n_blocks = n_chunks // nib

    def i_start(k, kb):
        pltpu.async_copy(src_hbm.at[t, pl.ds(k * nib, nib)], sblk.at[kb], sem_i.at[kb])
        pltpu.async_copy(dst_hbm.at[t, pl.ds(k * nib, nib)], dblk.at[kb], sem_i.at[kb])

    def i_wait(k, kb):
        pltpu.make_async_copy(src_hbm.at[t, pl.ds(k * nib, nib)], sblk.at[kb], sem_i.at[kb]).wait()
        pltpu.make_async_copy(dst_hbm.at[t, pl.ds(k * nib, nib)], dblk.at[kb], sem_i.at[kb]).wait()

    i_start(0, 0)
    init(c, r0, h0)
    plsc.subcore_barrier()

    def outer(k, carry):
        kb = lax.rem(k, 2)
        kn = lax.rem(k + 1, 2)
        i_wait(k, kb)

        @pl.when(k + 1 < n_blocks)
        def _():
            i_start(k + 1, kn)

        def g_start(q, b):
            pltpu.async_copy(hs_hbm.at[sblk.at[kb, q]], rows.at[b], sem_g.at[b])

        def g_wait(q, b):
            pltpu.make_async_copy(hs_hbm.at[sblk.at[kb, q]], rows.at[b], sem_g.at[b]).wait()

        def s_start(q, b):
            pltpu.async_copy(rows.at[b], acc.at[dblk.at[kb, q]], sem_s.at[b], add=True)

        def s_wait(q, b):
            pltpu.make_async_copy(rows.at[b], acc.at[dblk.at[kb, q]], sem_s.at[b]).wait()

        g_start(0, 0)

        def inner(q, carry2):
            b = lax.rem(q, 2)
            bn = lax.rem(q + 1, 2)

            @pl.when(q >= 1)
            def _():
                s_wait(q - 1, bn)

            @pl.when(q + 1 < nib)
            def _():
                g_start(q + 1, bn)

            g_wait(q, b)
            s_start(q, b)
            return carry2

        lax.fori_loop(0, nib, inner, 0)
        s_wait(nib - 1, (nib - 1) % 2)
        return carry

    lax.fori_loop(0, n_blocks, outer, 0)
    plsc.subcore_barrier()
    pltpu.sync_copy(acc.at[pl.ds(r0, ROWS_TILE)], out_hbm.at[pl.ds(h0 + r0, ROWS_TILE)])


def _agg_scratch(nib):
    return [
        pltpu.VMEM((2, nib, CHUNK), jnp.int32),
        pltpu.VMEM((2, nib, CHUNK), jnp.int32),
        pltpu.VMEM((2, CHUNK, CHUNK), _f32),
        pltpu.VMEM_SHARED((N_PAD, CHUNK), _f32),
        pltpu.SemaphoreType.DMA((2,)),
        pltpu.SemaphoreType.DMA((2,)),
        pltpu.SemaphoreType.DMA((2,)),
    ]


@functools.partial(
    pl.kernel,
    out_type=jax.ShapeDtypeStruct((2 * N_PAD, 128), _f32),
    mesh=_mesh(),
    scratch_types=_agg_scratch(NIB1),
)
def _agg_l1(src_hbm, dst_hbm, hs_hbm, out_hbm,
            sblk, dblk, rows, acc, sem_i, sem_g, sem_s):
    def init(c, r0, h0):
        pltpu.sync_copy(hs_hbm.at[pl.ds(h0 + r0, ROWS_TILE)], acc.at[pl.ds(r0, ROWS_TILE)])

    _agg_body(src_hbm, dst_hbm, hs_hbm, out_hbm, sblk, dblk, rows, acc,
              sem_i, sem_g, sem_s, N_CHUNKS, NIB1, init)


@functools.partial(
    pl.kernel,
    out_type=jax.ShapeDtypeStruct((2 * N_PAD, 128), _f32),
    mesh=_mesh(),
    scratch_types=_agg_scratch(NIB2),
)
def _agg_l2(src_hbm, dst_hbm, hs_hbm, zeros_hbm, out_hbm,
            sblk, dblk, rows, acc, sem_i, sem_g, sem_s):
    def init(c, r0, h0):
        for k in range(ROWS_TILE // CHUNK):
            pltpu.sync_copy(zeros_hbm, acc.at[pl.ds(r0 + k * CHUNK, CHUNK)])

    _agg_body(src_hbm, dst_hbm, hs_hbm, out_hbm, sblk, dblk, rows, acc,
              sem_i, sem_g, sem_s, N_CHUNKS2, NIB2, init)


def _tc_a_body(x_ref, w_ref, o_ref):
    o_ref[...] = jnp.dot(x_ref[...], w_ref[...], preferred_element_type=_f32)[None]


def _tc_s_body(h_ref, d0_ref, d1_ref, o_ref):
    dinv = lax.rsqrt(d0_ref[...] + d1_ref[...] + 1.0)
    o_ref[...] = h_ref[...] * dinv


def _tc_b_body(a_ref, b_ref, d0_ref, d1_ref, w_ref, b1_ref, o_ref):
    dinv = lax.rsqrt(d0_ref[...] + d1_ref[...] + 1.0)
    agg = jnp.concatenate([a_ref[...], b_ref[...]], axis=1)
    z = jnp.maximum(agg * dinv + b1_ref[...], 0.0)
    hs2 = jnp.dot(z, w_ref[...], preferred_element_type=_f32) * dinv
    o_ref[...] = jnp.broadcast_to(hs2[None], (2,) + hs2.shape)


def _tc_c_body(p0_ref, p1_ref, hs2_ref, d0_ref, d1_ref, b2_ref, o_ref):
    dinv = lax.rsqrt(d0_ref[...] + d1_ref[...] + 1.0)
    agg = p0_ref[...] + p1_ref[...] + hs2_ref[...]
    o_ref[...] = agg * dinv + b2_ref[...]


def _row_spec(cols):
    return pl.BlockSpec((ROW_BLK, cols), lambda i: (i, 0))


def _full_spec(rows, cols):
    return pl.BlockSpec((rows, cols), lambda i: (0, 0))


_GRID = (N_PAD // ROW_BLK,)

_tc_a = pl.pallas_call(
    _tc_a_body,
    grid=(N_PAD // ROW_BLK, 2),
    in_specs=[pl.BlockSpec((ROW_BLK, 256), lambda i, j: (i, 0)),
              pl.BlockSpec((256, 128), lambda i, j: (0, j))],
    out_specs=pl.BlockSpec((1, ROW_BLK, 128), lambda i, j: (j, i, 0)),
    out_shape=jax.ShapeDtypeStruct((2, N_PAD, 128), _f32),
)

_NBF = N_PAD // ROW_BLK

_tc_s = pl.pallas_call(
    _tc_s_body,
    grid=(2 * _NBF,),
    in_specs=[pl.BlockSpec((ROW_BLK, 128), lambda i: (i, 0)),
              pl.BlockSpec((ROW_BLK, 1), lambda i: (lax.rem(i, _NBF), 0)),
              pl.BlockSpec((ROW_BLK, 1), lambda i: (lax.rem(i, _NBF) + _NBF, 0))],
    out_specs=pl.BlockSpec((ROW_BLK, 128), lambda i: (i, 0)),
    out_shape=jax.ShapeDtypeStruct((2 * N_PAD, 128), _f32),
)

_NB = N_PAD // ROW_BLK

_tc_b = pl.pallas_call(
    _tc_b_body,
    grid=_GRID,
    in_specs=[pl.BlockSpec((ROW_BLK, 128), lambda i: (i, 0)),
              pl.BlockSpec((ROW_BLK, 128), lambda i: (i + _NB, 0)),
              pl.BlockSpec((ROW_BLK, 1), lambda i: (i, 0)),
              pl.BlockSpec((ROW_BLK, 1), lambda i: (i + _NB, 0)),
              _full_spec(256, 128), _full_spec(1, 256)],
    out_specs=pl.BlockSpec((2, ROW_BLK, 128), lambda i: (0, i, 0)),
    out_shape=jax.ShapeDtypeStruct((2, N_PAD, 128), _f32),
)

_tc_c = pl.pallas_call(
    _tc_c_body,
    grid=_GRID,
    in_specs=[pl.BlockSpec((ROW_BLK, 128), lambda i: (i, 0)),
              pl.BlockSpec((ROW_BLK, 128), lambda i: (i + _NB, 0)),
              pl.BlockSpec((ROW_BLK, 128), lambda i: (i, 0)),
              pl.BlockSpec((ROW_BLK, 1), lambda i: (i, 0)),
              pl.BlockSpec((ROW_BLK, 1), lambda i: (i + _NB, 0)),
              _full_spec(1, 128)],
    out_specs=_row_spec(128),
    out_shape=jax.ShapeDtypeStruct((N_PAD, 128), _f32),
)


def _pad_edges(v, n_split, e_tile, e_pad, n_chunks):
    return jnp.pad(v.reshape(n_split, e_tile), ((0, 0), (0, e_pad - e_tile)),
                   constant_values=N_NODES).reshape(n_split, n_chunks, CHUNK)


def kernel(x, edge_index, W1, b1, W2, b2):
    src = edge_index[0].astype(jnp.int32)
    dst = edge_index[1].astype(jnp.int32)
    src3 = _pad_edges(src, N_TILES, E_TILE, E_TILE_PAD, N_CHUNKS)
    dst3 = _pad_edges(dst, N_TILES, E_TILE, E_TILE_PAD, N_CHUNKS)
    srcl1 = jnp.concatenate([src3, src3 + N_PAD], axis=0)
    dstl1 = jnp.concatenate([dst3, dst3], axis=0)
    src32 = _pad_edges(src, 32, E_TILE2, E_TILE2_PAD, N_CHUNKS2)
    dst32 = _pad_edges(dst, 32, E_TILE2, E_TILE2_PAD, N_CHUNKS2)
    src32o = jnp.concatenate([src32[:N_TILES], src32[N_TILES:] + N_PAD], axis=0)

    ones128 = jnp.ones((CHUNK, CHUNK), _f32)
    zeros128 = jnp.zeros((CHUNK, CHUNK), _f32)
    degc = _deg_kernel(dst32, ones128, zeros128)[:, :1]

    h1 = _tc_a(x, W1).reshape(2 * N_PAD, 128)
    hs1 = _tc_s(h1, degc, degc)
    agg1 = _agg_l1(srcl1, dstl1, hs1)
    hs2cat = _tc_b(agg1, agg1, degc, degc, W2, b1.reshape(1, 256)).reshape(2 * N_PAD, 128)
    p = _agg_l2(src32o, dst32, hs2cat, zeros128)
    out = _tc_c(p, p, hs2cat, degc, degc, b2.reshape(1, 128))
    return out[:N_NODES]


# speedup vs baseline: 10.1287x; 1.0330x over previous
import functools

import jax
import jax.numpy as jnp
from jax import lax
from jax.experimental import pallas as pl
from jax.experimental.pallas import tpu as pltpu
from jax.experimental.pallas import tpu_sc as plsc

N_NODES = 10000
N_PAD = 10240
N_EDGES = 160000
N_TILES = 16
CHUNK = 128
NBUF = 4
N_CHUNKS = 80
E_TILE = N_EDGES // N_TILES
E_TILE_PAD = N_CHUNKS * CHUNK
N_CHUNKS2 = 40
E_TILE2 = N_EDGES // 32
E_TILE2_PAD = N_CHUNKS2 * CHUNK
ROWS_TILE = N_PAD // N_TILES
ROW_BLK = 512

_f32 = jnp.float32


def _mesh():
    return plsc.VectorSubcoreMesh(core_axis_name="c", subcore_axis_name="s")


NIB1 = 16
NIB2 = 8


@functools.partial(
    pl.kernel,
    out_type=jax.ShapeDtypeStruct((2 * N_PAD, CHUNK), _f32),
    mesh=_mesh(),
    scratch_types=[
        pltpu.VMEM((N_CHUNKS2, CHUNK), jnp.int32),
        pltpu.VMEM((CHUNK, CHUNK), _f32),
        pltpu.VMEM_SHARED((N_PAD, CHUNK), _f32),
        pltpu.SemaphoreType.DMA((NBUF,)),
    ],
)
def _deg_kernel(dst_hbm, ones_hbm, zeros_hbm, deg_hbm, dst_v, ones_v, acc, sem):
    c = lax.axis_index("c")
    s = lax.axis_index("s")
    t = c * N_TILES + s
    r0 = s * ROWS_TILE
    h0 = c * N_PAD

    pltpu.sync_copy(dst_hbm.at[t], dst_v)
    pltpu.sync_copy(ones_hbm, ones_v)
    for k in range(ROWS_TILE // CHUNK):
        pltpu.sync_copy(zeros_hbm, acc.at[pl.ds(r0 + k * CHUNK, CHUNK)])
    plsc.subcore_barrier()

    def body(i, carry):
        for b in range(NBUF):
            pltpu.async_copy(ones_v, acc.at[dst_v.at[i * NBUF + b]], sem.at[b], add=True)
        for b in range(NBUF):
            pltpu.make_async_copy(ones_v, acc.at[dst_v.at[i * NBUF + b]], sem.at[b]).wait()
        return carry

    lax.fori_loop(0, N_CHUNKS2 // NBUF, body, 0)
    plsc.subcore_barrier()
    pltpu.sync_copy(acc.at[pl.ds(r0, ROWS_TILE)], deg_hbm.at[pl.ds(h0 + r0, ROWS_TILE)])


def _agg_body(src_hbm, dst_hbm, hs_hbm, out_hbm, sblk, dblk, rows, acc,
              sem_i, sem_g, sem_s, n_chunks, nib, init, dst_16way=False):
    c = lax.axis_index("c")
    s = lax.axis_index("s")
    t = c * N_TILES + s
    r0 = s * ROWS_TILE
    h0 = c * N_PAD
    n_blocks = n_chunks // nib

    td = s if dst_16way else t

    def i_start(k, kb):
        pltpu.async_copy(src_hbm.at[t, pl.ds(k * nib, nib)], sblk.at[kb], sem_i.at[kb])
        pltpu.async_copy(dst_hbm.at[td, pl.ds(k * nib, nib)], dblk.at[kb], sem_i.at[kb])

    def i_wait(k, kb):
        pltpu.make_async_copy(src_hbm.at[t, pl.ds(k * nib, nib)], sblk.at[kb], sem_i.at[kb]).wait()
        pltpu.make_async_copy(dst_hbm.at[td, pl.ds(k * nib, nib)], dblk.at[kb], sem_i.at[kb]).wait()

    i_start(0, 0)
    init(c, r0, h0)
    plsc.subcore_barrier()

    def outer(k, carry):
        kb = lax.rem(k, 2)
        kn = lax.rem(k + 1, 2)
        i_wait(k, kb)

        @pl.when(k + 1 < n_blocks)
        def _():
            i_start(k + 1, kn)

        def g_start(q, b):
            pltpu.async_copy(hs_hbm.at[sblk.at[kb, q]], rows.at[b], sem_g.at[b])

        def g_wait(q, b):
            pltpu.make_async_copy(hs_hbm.at[sblk.at[kb, q]], rows.at[b], sem_g.at[b]).wait()

        def s_start(q, b):
            pltpu.async_copy(rows.at[b], acc.at[dblk.at[kb, q]], sem_s.at[b], add=True)

        def s_wait(q, b):
            pltpu.make_async_copy(rows.at[b], acc.at[dblk.at[kb, q]], sem_s.at[b]).wait()

        g_start(0, 0)

        def inner(q, carry2):
            b = lax.rem(q, 2)
            bn = lax.rem(q + 1, 2)

            @pl.when(q >= 1)
            def _():
                s_wait(q - 1, bn)

            @pl.when(q + 1 < nib)
            def _():
                g_start(q + 1, bn)

            g_wait(q, b)
            s_start(q, b)
            return carry2

        lax.fori_loop(0, nib, inner, 0)
        s_wait(nib - 1, (nib - 1) % 2)
        return carry

    lax.fori_loop(0, n_blocks, outer, 0)
    plsc.subcore_barrier()
    pltpu.sync_copy(acc.at[pl.ds(r0, ROWS_TILE)], out_hbm.at[pl.ds(h0 + r0, ROWS_TILE)])


def _agg_scratch(nib):
    return [
        pltpu.VMEM((2, nib, CHUNK), jnp.int32),
        pltpu.VMEM((2, nib, CHUNK), jnp.int32),
        pltpu.VMEM((2, CHUNK, CHUNK), _f32),
        pltpu.VMEM_SHARED((N_PAD, CHUNK), _f32),
        pltpu.SemaphoreType.DMA((2,)),
        pltpu.SemaphoreType.DMA((2,)),
        pltpu.SemaphoreType.DMA((2,)),
    ]


@functools.partial(
    pl.kernel,
    out_type=jax.ShapeDtypeStruct((2 * N_PAD, 128), _f32),
    mesh=_mesh(),
    scratch_types=_agg_scratch(NIB1),
)
def _agg_l1(src_hbm, dst_hbm, hs_hbm, out_hbm,
            sblk, dblk, rows, acc, sem_i, sem_g, sem_s):
    def init(c, r0, h0):
        pltpu.sync_copy(hs_hbm.at[pl.ds(h0 + r0, ROWS_TILE)], acc.at[pl.ds(r0, ROWS_TILE)])

    _agg_body(src_hbm, dst_hbm, hs_hbm, out_hbm, sblk, dblk, rows, acc,
              sem_i, sem_g, sem_s, N_CHUNKS, NIB1, init, dst_16way=True)


@functools.partial(
    pl.kernel,
    out_type=jax.ShapeDtypeStruct((2 * N_PAD, 128), _f32),
    mesh=_mesh(),
    scratch_types=_agg_scratch(NIB2),
)
def _agg_l2(src_hbm, dst_hbm, hs_hbm, zeros_hbm, out_hbm,
            sblk, dblk, rows, acc, sem_i, sem_g, sem_s):
    def init(c, r0, h0):
        for k in range(ROWS_TILE // CHUNK):
            pltpu.sync_copy(zeros_hbm, acc.at[pl.ds(r0 + k * CHUNK, CHUNK)])

    _agg_body(src_hbm, dst_hbm, hs_hbm, out_hbm, sblk, dblk, rows, acc,
              sem_i, sem_g, sem_s, N_CHUNKS2, NIB2, init)


def _tc_a_body(x_ref, w_ref, o_ref):
    h = jnp.dot(x_ref[...], w_ref[...], preferred_element_type=_f32)
    o_ref[...] = jnp.stack([h[:, :128], h[:, 128:]])


def _tc_s_body(h_ref, d0_ref, d1_ref, o_ref):
    dinv = lax.rsqrt(d0_ref[...] + d1_ref[...] + 1.0)
    o_ref[...] = h_ref[...] * dinv


def _tc_b_body(a_ref, b_ref, d0_ref, d1_ref, w_ref, b1_ref, o_ref):
    dinv = lax.rsqrt(d0_ref[...] + d1_ref[...] + 1.0)
    agg = jnp.concatenate([a_ref[...], b_ref[...]], axis=1)
    z = jnp.maximum(agg * dinv + b1_ref[...], 0.0)
    hs2 = jnp.dot(z, w_ref[...], preferred_element_type=_f32) * dinv
    o_ref[...] = jnp.broadcast_to(hs2[None], (2,) + hs2.shape)


def _tc_c_body(p0_ref, p1_ref, hs2_ref, d0_ref, d1_ref, b2_ref, o_ref):
    dinv = lax.rsqrt(d0_ref[...] + d1_ref[...] + 1.0)
    agg = p0_ref[...] + p1_ref[...] + hs2_ref[...]
    o_ref[...] = agg * dinv + b2_ref[...]


def _row_spec(cols):
    return pl.BlockSpec((ROW_BLK, cols), lambda i: (i, 0))


def _full_spec(rows, cols):
    return pl.BlockSpec((rows, cols), lambda i: (0, 0))


_GRID = (N_PAD // ROW_BLK,)

_tc_a = pl.pallas_call(
    _tc_a_body,
    grid=(N_PAD // ROW_BLK,),
    in_specs=[pl.BlockSpec((ROW_BLK, 256), lambda i: (i, 0)),
              pl.BlockSpec((256, 256), lambda i: (0, 0))],
    out_specs=pl.BlockSpec((2, ROW_BLK, 128), lambda i: (0, i, 0)),
    out_shape=jax.ShapeDtypeStruct((2, N_PAD, 128), _f32),
)

_NBF = N_PAD // ROW_BLK
_SBLK = 2048
_NSB = N_PAD // _SBLK

_tc_s = pl.pallas_call(
    _tc_s_body,
    grid=(2 * _NSB,),
    in_specs=[pl.BlockSpec((_SBLK, 128), lambda i: (i, 0)),
              pl.BlockSpec((_SBLK, 1), lambda i: (lax.rem(i, _NSB), 0)),
              pl.BlockSpec((_SBLK, 1), lambda i: (lax.rem(i, _NSB) + _NSB, 0))],
    out_specs=pl.BlockSpec((_SBLK, 128), lambda i: (i, 0)),
    out_shape=jax.ShapeDtypeStruct((2 * N_PAD, 128), _f32),
)

_NB = N_PAD // ROW_BLK

_tc_b = pl.pallas_call(
    _tc_b_body,
    grid=_GRID,
    in_specs=[pl.BlockSpec((ROW_BLK, 128), lambda i: (i, 0)),
              pl.BlockSpec((ROW_BLK, 128), lambda i: (i + _NB, 0)),
              pl.BlockSpec((ROW_BLK, 1), lambda i: (i, 0)),
              pl.BlockSpec((ROW_BLK, 1), lambda i: (i + _NB, 0)),
              _full_spec(256, 128), _full_spec(1, 256)],
    out_specs=pl.BlockSpec((2, ROW_BLK, 128), lambda i: (0, i, 0)),
    out_shape=jax.ShapeDtypeStruct((2, N_PAD, 128), _f32),
)

_tc_c = pl.pallas_call(
    _tc_c_body,
    grid=_GRID,
    in_specs=[pl.BlockSpec((ROW_BLK, 128), lambda i: (i, 0)),
              pl.BlockSpec((ROW_BLK, 128), lambda i: (i + _NB, 0)),
              pl.BlockSpec((ROW_BLK, 128), lambda i: (i, 0)),
              pl.BlockSpec((ROW_BLK, 1), lambda i: (i, 0)),
              pl.BlockSpec((ROW_BLK, 1), lambda i: (i + _NB, 0)),
              _full_spec(1, 128)],
    out_specs=_row_spec(128),
    out_shape=jax.ShapeDtypeStruct((N_NODES, 128), _f32),
)


def _pad_edges(v, n_split, e_tile, e_pad, n_chunks):
    return jnp.pad(v.reshape(n_split, e_tile), ((0, 0), (0, e_pad - e_tile)),
                   constant_values=N_NODES).reshape(n_split, n_chunks, CHUNK)


def kernel(x, edge_index, W1, b1, W2, b2):
    src = edge_index[0].astype(jnp.int32)
    dst = edge_index[1].astype(jnp.int32)
    src3 = _pad_edges(src, N_TILES, E_TILE, E_TILE_PAD, N_CHUNKS)
    dst3 = _pad_edges(dst, N_TILES, E_TILE, E_TILE_PAD, N_CHUNKS)
    srcl1 = jnp.concatenate([src3, src3 + N_PAD], axis=0)
    src32 = _pad_edges(src, 32, E_TILE2, E_TILE2_PAD, N_CHUNKS2)
    dst32 = _pad_edges(dst, 32, E_TILE2, E_TILE2_PAD, N_CHUNKS2)
    src32o = jnp.concatenate([src32[:N_TILES], src32[N_TILES:] + N_PAD], axis=0)

    ones128 = jnp.ones((CHUNK, CHUNK), _f32)
    zeros128 = jnp.zeros((CHUNK, CHUNK), _f32)
    degc = _deg_kernel(dst32, ones128, zeros128)[:, :1]

    h1 = _tc_a(x, W1).reshape(2 * N_PAD, 128)
    hs1 = _tc_s(h1, degc, degc)
    agg1 = _agg_l1(srcl1, dst3, hs1)
    hs2cat = _tc_b(agg1, agg1, degc, degc, W2, b1.reshape(1, 256)).reshape(2 * N_PAD, 128)
    p = _agg_l2(src32o, dst32, hs2cat, zeros128)
    return _tc_c(p, p, hs2cat, degc, degc, b2.reshape(1, 128))


# speedup vs baseline: 10.2966x; 1.0166x over previous
import functools

import jax
import jax.numpy as jnp
from jax import lax
from jax.experimental import pallas as pl
from jax.experimental.pallas import tpu as pltpu
from jax.experimental.pallas import tpu_sc as plsc

N_NODES = 10000
N_PAD = 10240
N_EDGES = 160000
N_TILES = 16
CHUNK = 128
NBUF = 4
N_CHUNKS = 80
E_TILE = N_EDGES // N_TILES
E_TILE_PAD = N_CHUNKS * CHUNK
N_CHUNKS2 = 40
E_TILE2 = N_EDGES // 32
E_TILE2_PAD = N_CHUNKS2 * CHUNK
ROWS_TILE = N_PAD // N_TILES
ROW_BLK = 1024

_f32 = jnp.float32


def _mesh():
    return plsc.VectorSubcoreMesh(core_axis_name="c", subcore_axis_name="s")


NIB1 = 16
NIB2 = 8


@functools.partial(
    pl.kernel,
    out_type=jax.ShapeDtypeStruct((2 * N_PAD, CHUNK), _f32),
    mesh=_mesh(),
    scratch_types=[
        pltpu.VMEM((N_CHUNKS2, CHUNK), jnp.int32),
        pltpu.VMEM((CHUNK, CHUNK), _f32),
        pltpu.VMEM_SHARED((N_PAD, CHUNK), _f32),
        pltpu.SemaphoreType.DMA((NBUF,)),
    ],
)
def _deg_kernel(dst_hbm, ones_hbm, zeros_hbm, deg_hbm, dst_v, ones_v, acc, sem):
    c = lax.axis_index("c")
    s = lax.axis_index("s")
    t = c * N_TILES + s
    r0 = s * ROWS_TILE
    h0 = c * N_PAD

    pltpu.sync_copy(dst_hbm.at[t], dst_v)
    pltpu.sync_copy(ones_hbm, ones_v)
    for k in range(ROWS_TILE // CHUNK):
        pltpu.sync_copy(zeros_hbm, acc.at[pl.ds(r0 + k * CHUNK, CHUNK)])
    plsc.subcore_barrier()

    def body(i, carry):
        for b in range(NBUF):
            pltpu.async_copy(ones_v, acc.at[dst_v.at[i * NBUF + b]], sem.at[b], add=True)
        for b in range(NBUF):
            pltpu.make_async_copy(ones_v, acc.at[dst_v.at[i * NBUF + b]], sem.at[b]).wait()
        return carry

    lax.fori_loop(0, N_CHUNKS2 // NBUF, body, 0)
    plsc.subcore_barrier()
    pltpu.sync_copy(acc.at[pl.ds(r0, ROWS_TILE)], deg_hbm.at[pl.ds(h0 + r0, ROWS_TILE)])


def _agg_body(src_hbm, dst_hbm, hs_hbm, out_hbm, sblk, dblk, rows, acc,
              sem_i, sem_g, sem_s, n_chunks, nib, init, dst_16way=False):
    c = lax.axis_index("c")
    s = lax.axis_index("s")
    t = c * N_TILES + s
    r0 = s * ROWS_TILE
    h0 = c * N_PAD
    n_blocks = n_chunks // nib

    td = s if dst_16way else t

    def i_start(k, kb):
        pltpu.async_copy(src_hbm.at[t, pl.ds(k * nib, nib)], sblk.at[kb], sem_i.at[kb])
        pltpu.async_copy(dst_hbm.at[td, pl.ds(k * nib, nib)], dblk.at[kb], sem_i.at[kb])

    def i_wait(k, kb):
        pltpu.make_async_copy(src_hbm.at[t, pl.ds(k * nib, nib)], sblk.at[kb], sem_i.at[kb]).wait()
        pltpu.make_async_copy(dst_hbm.at[td, pl.ds(k * nib, nib)], dblk.at[kb], sem_i.at[kb]).wait()

    i_start(0, 0)
    init(c, r0, h0)
    plsc.subcore_barrier()

    def outer(k, carry):
        kb = lax.rem(k, 2)
        kn = lax.rem(k + 1, 2)
        i_wait(k, kb)

        @pl.when(k + 1 < n_blocks)
        def _():
            i_start(k + 1, kn)

        def g_start(q, b):
            pltpu.async_copy(hs_hbm.at[sblk.at[kb, q]], rows.at[b], sem_g.at[b])

        def g_wait(q, b):
            pltpu.make_async_copy(hs_hbm.at[sblk.at[kb, q]], rows.at[b], sem_g.at[b]).wait()

        def s_start(q, b):
            pltpu.async_copy(rows.at[b], acc.at[dblk.at[kb, q]], sem_s.at[b], add=True)

        def s_wait(q, b):
            pltpu.make_async_copy(rows.at[b], acc.at[dblk.at[kb, q]], sem_s.at[b]).wait()

        g_start(0, 0)

        def inner(q, carry2):
            b = lax.rem(q, 2)
            bn = lax.rem(q + 1, 2)

            @pl.when(q >= 1)
            def _():
                s_wait(q - 1, bn)

            @pl.when(q + 1 < nib)
            def _():
                g_start(q + 1, bn)

            g_wait(q, b)
            s_start(q, b)
            return carry2

        lax.fori_loop(0, nib, inner, 0)
        s_wait(nib - 1, (nib - 1) % 2)
        return carry

    lax.fori_loop(0, n_blocks, outer, 0)
    plsc.subcore_barrier()
    pltpu.sync_copy(acc.at[pl.ds(r0, ROWS_TILE)], out_hbm.at[pl.ds(h0 + r0, ROWS_TILE)])


def _agg_scratch(nib):
    return [
        pltpu.VMEM((2, nib, CHUNK), jnp.int32),
        pltpu.VMEM((2, nib, CHUNK), jnp.int32),
        pltpu.VMEM((2, CHUNK, CHUNK), _f32),
        pltpu.VMEM_SHARED((N_PAD, CHUNK), _f32),
        pltpu.SemaphoreType.DMA((2,)),
        pltpu.SemaphoreType.DMA((2,)),
        pltpu.SemaphoreType.DMA((2,)),
    ]


@functools.partial(
    pl.kernel,
    out_type=jax.ShapeDtypeStruct((2 * N_PAD, 128), _f32),
    mesh=_mesh(),
    scratch_types=_agg_scratch(NIB1),
)
def _agg_l1(src_hbm, dst_hbm, hs_hbm, out_hbm,
            sblk, dblk, rows, acc, sem_i, sem_g, sem_s):
    def init(c, r0, h0):
        pltpu.sync_copy(hs_hbm.at[pl.ds(h0 + r0, ROWS_TILE)], acc.at[pl.ds(r0, ROWS_TILE)])

    _agg_body(src_hbm, dst_hbm, hs_hbm, out_hbm, sblk, dblk, rows, acc,
              sem_i, sem_g, sem_s, N_CHUNKS, NIB1, init, dst_16way=True)


@functools.partial(
    pl.kernel,
    out_type=jax.ShapeDtypeStruct((2 * N_PAD, 128), _f32),
    mesh=_mesh(),
    scratch_types=_agg_scratch(NIB2),
)
def _agg_l2(src_hbm, dst_hbm, hs_hbm, zeros_hbm, out_hbm,
            sblk, dblk, rows, acc, sem_i, sem_g, sem_s):
    def init(c, r0, h0):
        for k in range(ROWS_TILE // CHUNK):
            pltpu.sync_copy(zeros_hbm, acc.at[pl.ds(r0 + k * CHUNK, CHUNK)])

    _agg_body(src_hbm, dst_hbm, hs_hbm, out_hbm, sblk, dblk, rows, acc,
              sem_i, sem_g, sem_s, N_CHUNKS2, NIB2, init)


def _tc_a_body(x_ref, w_ref, o_ref):
    h = jnp.dot(x_ref[...], w_ref[...], preferred_element_type=_f32)
    o_ref[...] = jnp.stack([h[:, :128], h[:, 128:]])


def _tc_s_body(h_ref, d0_ref, d1_ref, o_ref):
    dinv = lax.rsqrt(d0_ref[...] + d1_ref[...] + 1.0)
    o_ref[...] = h_ref[...] * dinv


def _tc_b_body(a_ref, b_ref, d0_ref, d1_ref, w_ref, b1_ref, o_ref):
    dinv = lax.rsqrt(d0_ref[...] + d1_ref[...] + 1.0)
    agg = jnp.concatenate([a_ref[...], b_ref[...]], axis=1)
    z = jnp.maximum(agg * dinv + b1_ref[...], 0.0)
    hs2 = jnp.dot(z, w_ref[...], preferred_element_type=_f32) * dinv
    o_ref[...] = jnp.broadcast_to(hs2[None], (2,) + hs2.shape)


def _tc_c_body(p0_ref, p1_ref, hs2_ref, d0_ref, d1_ref, b2_ref, o_ref):
    dinv = lax.rsqrt(d0_ref[...] + d1_ref[...] + 1.0)
    agg = p0_ref[...] + p1_ref[...] + hs2_ref[...]
    o_ref[...] = agg * dinv + b2_ref[...]


def _row_spec(cols):
    return pl.BlockSpec((ROW_BLK, cols), lambda i: (i, 0))


def _full_spec(rows, cols):
    return pl.BlockSpec((rows, cols), lambda i: (0, 0))


_GRID = (N_PAD // ROW_BLK,)

_tc_a = pl.pallas_call(
    _tc_a_body,
    grid=(N_PAD // ROW_BLK,),
    in_specs=[pl.BlockSpec((ROW_BLK, 256), lambda i: (i, 0)),
              pl.BlockSpec((256, 256), lambda i: (0, 0))],
    out_specs=pl.BlockSpec((2, ROW_BLK, 128), lambda i: (0, i, 0)),
    out_shape=jax.ShapeDtypeStruct((2, N_PAD, 128), _f32),
)

_NBF = N_PAD // ROW_BLK
_SBLK = 2048
_NSB = N_PAD // _SBLK

_tc_s = pl.pallas_call(
    _tc_s_body,
    grid=(2 * _NSB,),
    in_specs=[pl.BlockSpec((_SBLK, 128), lambda i: (i, 0)),
              pl.BlockSpec((_SBLK, 1), lambda i: (lax.rem(i, _NSB), 0)),
              pl.BlockSpec((_SBLK, 1), lambda i: (lax.rem(i, _NSB) + _NSB, 0))],
    out_specs=pl.BlockSpec((_SBLK, 128), lambda i: (i, 0)),
    out_shape=jax.ShapeDtypeStruct((2 * N_PAD, 128), _f32),
)

_NB = N_PAD // ROW_BLK

_tc_b = pl.pallas_call(
    _tc_b_body,
    grid=_GRID,
    in_specs=[pl.BlockSpec((ROW_BLK, 128), lambda i: (i, 0)),
              pl.BlockSpec((ROW_BLK, 128), lambda i: (i + _NB, 0)),
              pl.BlockSpec((ROW_BLK, 1), lambda i: (i, 0)),
              pl.BlockSpec((ROW_BLK, 1), lambda i: (i + _NB, 0)),
              _full_spec(256, 128), _full_spec(1, 256)],
    out_specs=pl.BlockSpec((2, ROW_BLK, 128), lambda i: (0, i, 0)),
    out_shape=jax.ShapeDtypeStruct((2, N_PAD, 128), _f32),
)

_tc_c = pl.pallas_call(
    _tc_c_body,
    grid=_GRID,
    in_specs=[pl.BlockSpec((ROW_BLK, 128), lambda i: (i, 0)),
              pl.BlockSpec((ROW_BLK, 128), lambda i: (i + _NB, 0)),
              pl.BlockSpec((ROW_BLK, 128), lambda i: (i, 0)),
              pl.BlockSpec((ROW_BLK, 1), lambda i: (i, 0)),
              pl.BlockSpec((ROW_BLK, 1), lambda i: (i + _NB, 0)),
              _full_spec(1, 128)],
    out_specs=_row_spec(128),
    out_shape=jax.ShapeDtypeStruct((N_NODES, 128), _f32),
)


def _pad_edges(v, n_split, e_tile, e_pad, n_chunks):
    return jnp.pad(v.reshape(n_split, e_tile), ((0, 0), (0, e_pad - e_tile)),
                   constant_values=N_NODES).reshape(n_split, n_chunks, CHUNK)


def kernel(x, edge_index, W1, b1, W2, b2):
    src = edge_index[0].astype(jnp.int32)
    dst = edge_index[1].astype(jnp.int32)
    src3 = _pad_edges(src, N_TILES, E_TILE, E_TILE_PAD, N_CHUNKS)
    dst3 = _pad_edges(dst, N_TILES, E_TILE, E_TILE_PAD, N_CHUNKS)
    srcl1 = jnp.concatenate([src3, src3 + N_PAD], axis=0)
    src32 = _pad_edges(src, 32, E_TILE2, E_TILE2_PAD, N_CHUNKS2)
    dst32 = _pad_edges(dst, 32, E_TILE2, E_TILE2_PAD, N_CHUNKS2)
    src32o = jnp.concatenate([src32[:N_TILES], src32[N_TILES:] + N_PAD], axis=0)

    ones128 = jnp.ones((CHUNK, CHUNK), _f32)
    zeros128 = jnp.zeros((CHUNK, CHUNK), _f32)
    degc = _deg_kernel(dst32, ones128, zeros128)[:, :1]

    h1 = _tc_a(x, W1).reshape(2 * N_PAD, 128)
    hs1 = _tc_s(h1, degc, degc)
    agg1 = _agg_l1(srcl1, dst3, hs1)
    hs2cat = _tc_b(agg1, agg1, degc, degc, W2, b1.reshape(1, 256)).reshape(2 * N_PAD, 128)
    p = _agg_l2(src32o, dst32, hs2cat, zeros128)
    return _tc_c(p, p, hs2cat, degc, degc, b2.reshape(1, 128))
